# baseline (speedup 1.0000x reference)
import jax
import jax.numpy as jnp
from jax import lax
from jax.experimental import pallas as pl
from jax.experimental.pallas import tpu as pltpu
from jax.experimental.pallas import tpu_sc as plsc

_INFO = plsc.get_sparse_core_info()
_NC = _INFO.num_cores
_NS = _INFO.num_subcores
_NW = _NC * _NS
_L = _INFO.num_lanes

_E = 320000
_D = 128
_C = 80
_PER_W = _E // _NW
_NCHUNK = _PER_W // _C


def _vperm(x, idx):
    return lax.gather(
        x, idx[:, None],
        lax.GatherDimensionNumbers(
            offset_dims=(), collapsed_slice_dims=(0,), start_index_map=(0,)),
        slice_sizes=(1,),
        mode=lax.GatherScatterMode.PROMISE_IN_BOUNDS)


def _sc_body(elf_hbm, mq_hbm, sq_hbm, out_hbm,
             idx_u_all, idx_m_all, ru0, rm0, ru1, rm1, ov0, ov1,
             sg0, sg1, so0, so1):
    wid = lax.axis_index("s") * _NC + lax.axis_index("c")
    wbase = wid * _PER_W

    ru = (ru0, ru1)
    rm = (rm0, rm1)
    ov = (ov0, ov1)
    sg = (sg0, sg1)
    so = (so0, so1)

    pltpu.sync_copy(elf_hbm.at[pl.ds(wbase, _PER_W)], idx_u_all)
    pltpu.sync_copy(elf_hbm.at[pl.ds(_E + wbase, _PER_W)], idx_m_all)

    def gather_descs(c, p):
        cu = pltpu.make_async_copy(
            mq_hbm.at[idx_u_all.at[pl.ds(c * _C, _C)]], ru[p], sg[p])
        cm = pltpu.make_async_copy(
            sq_hbm.at[idx_m_all.at[pl.ds(c * _C, _C)]], rm[p], sg[p])
        return cu, cm

    def start_gathers(c, p):
        cu, cm = gather_descs(c, p)
        cu.start()
        cm.start()

    def wait_gathers(c, p):
        cu, cm = gather_descs(c, p)
        cu.wait()
        cm.wait()

    lane = lax.iota(jnp.int32, _L)
    rots = [(lane + r) & (_L - 1) for r in (8, 4, 2, 1)]
    himask = jnp.full((_L,), -65536, jnp.int32)

    def compute_chunk(p):
        rup, rmp, ovp = ru[p], rm[p], ov[p]

        def group_body(g, _):
            res = jnp.zeros((_L,), jnp.float32)
            for j in range(_L):
                i = g * _L + j
                acc_hi = acc_lo = None
                for k in range(_D // (2 * _L)):
                    ub = plsc.bitcast(rup[i, pl.ds(k * _L, _L)], jnp.bfloat16)
                    mb = plsc.bitcast(rmp[i, pl.ds(k * _L, _L)], jnp.bfloat16)
                    pi = plsc.bitcast(ub * mb, jnp.int32)
                    hi = plsc.bitcast(pi, jnp.float32)
                    lo = plsc.bitcast(pi << 16, jnp.float32)
                    if acc_hi is None:
                        acc_hi, acc_lo = hi, lo
                    else:
                        acc_hi = acc_hi + hi
                        acc_lo = acc_lo + lo
                acc = acc_hi + acc_lo
                for rot in rots:
                    acc = acc + _vperm(acc, rot)
                res = jnp.where(lane == j, acc, res)
            ovp[pl.ds(g * _L, _L)] = res
            return _

        lax.fori_loop(0, _C // _L, group_body, None)

    def out_desc(c, p):
        return pltpu.make_async_copy(
            ov[p], out_hbm.at[pl.ds(wbase + c * _C, _C)], so[p])

    start_gathers(0, 0)
    start_gathers(1, 1)

    def loop_body(t, _):
        for p in (0, 1):
            c = 2 * t + p
            wait_gathers(c, p)

            @pl.when(c >= 2)
            def _wait_prev_out():
                out_desc(c - 2, p).wait()

            compute_chunk(p)
            out_desc(c, p).start()

            @pl.when(c + 2 < _NCHUNK)
            def _start_next():
                start_gathers(c + 2, p)
        return _

    lax.fori_loop(0, (_NCHUNK - 1) // 2, loop_body, None)

    c_last = _NCHUNK - 1
    wait_gathers(c_last, 0)
    out_desc(c_last - 2, 0).wait()
    compute_chunk(0)
    out_desc(c_last, 0).start()
    out_desc(c_last - 1, 1).wait()
    out_desc(c_last, 0).wait()


@jax.jit
def _classifier_sc(el, mq, sq):
    mesh = plsc.VectorSubcoreMesh(core_axis_name="c", subcore_axis_name="s")
    fn = pl.kernel(
        _sc_body,
        out_type=jax.ShapeDtypeStruct((_E,), jnp.float32),
        mesh=mesh,
        compiler_params=pltpu.CompilerParams(needs_layout_passes=False, use_tc_tiling_on_sc=False),
        scratch_types=[
            pltpu.VMEM((_PER_W,), jnp.int32),
            pltpu.VMEM((_PER_W,), jnp.int32),
            pltpu.VMEM((_C, _D // 2), jnp.int32),
            pltpu.VMEM((_C, _D // 2), jnp.int32),
            pltpu.VMEM((_C, _D // 2), jnp.int32),
            pltpu.VMEM((_C, _D // 2), jnp.int32),
            pltpu.VMEM((_C,), jnp.float32),
            pltpu.VMEM((_C,), jnp.float32),
            pltpu.SemaphoreType.DMA,
            pltpu.SemaphoreType.DMA,
            pltpu.SemaphoreType.DMA,
            pltpu.SemaphoreType.DMA,
        ],
    )
    return fn(el, mq, sq)


def _pack_table(t):
    x = lax.bitcast_convert_type(t, jnp.int32)
    r = x + 0x7FFF + ((x >> 16) & 1)
    return (r[:, :_D // 2] & -65536) | ((r[:, _D // 2:] >> 16) & 65535)


def kernel(mq, sq, edge_label_index):
    return _classifier_sc(edge_label_index.reshape(-1),
                          _pack_table(mq), _pack_table(sq))


# speedup vs baseline: 1.2828x; 1.2828x over previous
import jax
import jax.numpy as jnp
from jax import lax
from jax.experimental import pallas as pl
from jax.experimental.pallas import tpu as pltpu
from jax.experimental.pallas import tpu_sc as plsc

_INFO = plsc.get_sparse_core_info()
_NC = _INFO.num_cores
_NS = _INFO.num_subcores
_NW = _NC * _NS
_L = _INFO.num_lanes

_E = 320000
_V = 10000
_D = 128
_C = 80
_PER_W = _E // _NW
_NCHUNK = _PER_W // _C


def _vperm(x, idx):
    return lax.gather(
        x, idx[:, None],
        lax.GatherDimensionNumbers(
            offset_dims=(), collapsed_slice_dims=(0,), start_index_map=(0,)),
        slice_sizes=(1,),
        mode=lax.GatherScatterMode.PROMISE_IN_BOUNDS)


def _sc_body(elf_hbm, mq_hbm, sq_hbm, out_hbm,
             mq_s, sq_s, idx_u_all, idx_m_all, ru0, rm0, ru1, rm1, ov0, ov1,
             sg0, sg1, so0, so1, st0, st1):
    sid = lax.axis_index("s")
    wid = sid * _NC + lax.axis_index("c")
    wbase = wid * _PER_W

    ru = (ru0, ru1)
    rm = (rm0, rm1)
    ov = (ov0, ov1)
    sg = (sg0, sg1)
    so = (so0, so1)

    rows_per_sub = _V // _NS
    tu = pltpu.make_async_copy(
        mq_hbm.at[pl.ds(sid * rows_per_sub, rows_per_sub)],
        mq_s.at[pl.ds(sid * rows_per_sub, rows_per_sub)], st0)
    tm = pltpu.make_async_copy(
        sq_hbm.at[pl.ds(sid * rows_per_sub, rows_per_sub)],
        sq_s.at[pl.ds(sid * rows_per_sub, rows_per_sub)], st1)
    tu.start()
    tm.start()

    pltpu.sync_copy(elf_hbm.at[pl.ds(wbase, _PER_W)], idx_u_all)
    pltpu.sync_copy(elf_hbm.at[pl.ds(_E + wbase, _PER_W)], idx_m_all)

    tu.wait()
    tm.wait()
    plsc.subcore_barrier()

    def gather_descs(c, p):
        cu = pltpu.make_async_copy(
            mq_s.at[idx_u_all.at[pl.ds(c * _C, _C)]], ru[p], sg[p])
        cm = pltpu.make_async_copy(
            sq_s.at[idx_m_all.at[pl.ds(c * _C, _C)]], rm[p], sg[p])
        return cu, cm

    def start_gathers(c, p):
        cu, cm = gather_descs(c, p)
        cu.start()
        cm.start()

    def wait_gathers(c, p):
        cu, cm = gather_descs(c, p)
        cu.wait()
        cm.wait()

    lane = lax.iota(jnp.int32, _L)
    rots = [(lane + r) & (_L - 1) for r in (8, 4, 2, 1)]
    himask = jnp.full((_L,), -65536, jnp.int32)

    def compute_chunk(p):
        rup, rmp, ovp = ru[p], rm[p], ov[p]

        def group_body(g, _):
            res = jnp.zeros((_L,), jnp.float32)
            for j in range(_L):
                i = g * _L + j
                acc_hi = acc_lo = None
                for k in range(_D // (2 * _L)):
                    ub = plsc.bitcast(rup[i, pl.ds(k * _L, _L)], jnp.bfloat16)
                    mb = plsc.bitcast(rmp[i, pl.ds(k * _L, _L)], jnp.bfloat16)
                    pi = plsc.bitcast(ub * mb, jnp.int32)
                    hi = plsc.bitcast(pi & himask, jnp.float32)
                    lo = plsc.bitcast(pi << 16, jnp.float32)
                    if acc_hi is None:
                        acc_hi, acc_lo = hi, lo
                    else:
                        acc_hi = acc_hi + hi
                        acc_lo = acc_lo + lo
                acc = acc_hi + acc_lo
                for rot in rots:
                    acc = acc + _vperm(acc, rot)
                res = jnp.where(lane == j, acc, res)
            ovp[pl.ds(g * _L, _L)] = res
            return _

        lax.fori_loop(0, _C // _L, group_body, None)

    def out_desc(c, p):
        return pltpu.make_async_copy(
            ov[p], out_hbm.at[pl.ds(wbase + c * _C, _C)], so[p])

    start_gathers(0, 0)
    start_gathers(1, 1)

    def loop_body(t, _):
        for p in (0, 1):
            c = 2 * t + p
            wait_gathers(c, p)

            @pl.when(c >= 2)
            def _wait_prev_out():
                out_desc(c - 2, p).wait()

            compute_chunk(p)
            out_desc(c, p).start()

            @pl.when(c + 2 < _NCHUNK)
            def _start_next():
                start_gathers(c + 2, p)
        return _

    lax.fori_loop(0, (_NCHUNK - 1) // 2, loop_body, None)

    c_last = _NCHUNK - 1
    wait_gathers(c_last, 0)
    out_desc(c_last - 2, 0).wait()
    compute_chunk(0)
    out_desc(c_last, 0).start()
    out_desc(c_last - 1, 1).wait()
    out_desc(c_last, 0).wait()


@jax.jit
def _classifier_sc(el, mq, sq):
    mesh = plsc.VectorSubcoreMesh(core_axis_name="c", subcore_axis_name="s")
    fn = pl.kernel(
        _sc_body,
        out_type=jax.ShapeDtypeStruct((_E,), jnp.float32),
        mesh=mesh,
        compiler_params=pltpu.CompilerParams(needs_layout_passes=False, use_tc_tiling_on_sc=False),
        scratch_types=[
            pltpu.VMEM_SHARED((_V, _D // 2), jnp.int32),
            pltpu.VMEM_SHARED((_V, _D // 2), jnp.int32),
            pltpu.VMEM((_PER_W,), jnp.int32),
            pltpu.VMEM((_PER_W,), jnp.int32),
            pltpu.VMEM((_C, _D // 2), jnp.int32),
            pltpu.VMEM((_C, _D // 2), jnp.int32),
            pltpu.VMEM((_C, _D // 2), jnp.int32),
            pltpu.VMEM((_C, _D // 2), jnp.int32),
            pltpu.VMEM((_C,), jnp.float32),
            pltpu.VMEM((_C,), jnp.float32),
            pltpu.SemaphoreType.DMA,
            pltpu.SemaphoreType.DMA,
            pltpu.SemaphoreType.DMA,
            pltpu.SemaphoreType.DMA,
            pltpu.SemaphoreType.DMA,
            pltpu.SemaphoreType.DMA,
        ],
    )
    return fn(el, mq, sq)


def _pack_table(t):
    x = lax.bitcast_convert_type(t, jnp.int32)
    r = x + 0x7FFF + ((x >> 16) & 1)
    return (r[:, :_D // 2] & -65536) | ((r[:, _D // 2:] >> 16) & 65535)


def kernel(mq, sq, edge_label_index):
    return _classifier_sc(edge_label_index.reshape(-1),
                          _pack_table(mq), _pack_table(sq))
